# baseline (speedup 1.0000x reference)
import functools

import jax
import jax.numpy as jnp
from jax import lax
from jax.experimental import pallas as pl
from jax.experimental.pallas import tpu as pltpu
from jax.experimental.pallas import tpu_sc as plsc

N_NODES = 10000
D_FEAT = 128
N_EDGES = 320000

_NC = 2
_NS = 16
_NW = _NC * _NS
_PER_W = N_EDGES // _NW
_C = 80
_NCHUNK = _PER_W // _C

_LANES = 16
_KVEC = D_FEAT // _LANES


def _sc_kernel(xd, xs, idx0, idx1, out,
               i0all, i1all, r0a, r1a, r0b, r1b, ov,
               s0a, s1a, s0b, s1b):
    wid = lax.axis_index("s") * _NC + lax.axis_index("c")
    pltpu.sync_copy(idx0.at[wid], i0all)
    pltpu.sync_copy(idx1.at[wid], i1all)

    lane = lax.iota(jnp.int32, _LANES)
    skew = [(lane + f0) & (_LANES - 1) for f0 in range(_LANES)]

    def issue(g, r0, r1, s0, s1):
        pltpu.async_copy(xd.at[i0all.at[g]], r0, s0)
        pltpu.async_copy(xs.at[i1all.at[g]], r1, s1)

    def wait(g, r0, r1, s0, s1):
        pltpu.make_async_copy(xd.at[i0all.at[g]], r0, s0).wait()
        pltpu.make_async_copy(xs.at[i1all.at[g]], r1, s1).wait()

    def compute(g, r0, r1):
        def group_body(gr, gcarry):
            rows = gr * _LANES + lane
            acc = [jnp.zeros((_LANES,), jnp.float32) for _ in range(8)]
            for blk in range(_KVEC):
                cb = jnp.full((_LANES,), blk * _LANES, jnp.int32)
                for f0 in range(_LANES):
                    col = skew[f0] + cb
                    a = plsc.load_gather(r0, [rows, col])
                    b = plsc.load_gather(r1, [rows, col])
                    acc[f0 & 7] = acc[f0 & 7] + a * b
            acc = [acc[2 * t] + acc[2 * t + 1] for t in range(4)]
            acc = [acc[2 * t] + acc[2 * t + 1] for t in range(2)]
            ov[pl.ds(g * _C + gr * _LANES, _LANES)] = acc[0] + acc[1]
            return gcarry

        lax.fori_loop(0, _C // _LANES, group_body, 0)

    issue(0, r0a, r1a, s0a, s1a)
    issue(1, r0b, r1b, s0b, s1b)

    def pair_body(i, carry):
        g = 2 * i
        wait(g, r0a, r1a, s0a, s1a)
        compute(g, r0a, r1a)
        issue(g + 2, r0a, r1a, s0a, s1a)
        wait(g + 1, r0b, r1b, s0b, s1b)
        compute(g + 1, r0b, r1b)

        @pl.when(g + 3 < _NCHUNK)
        def _():
            issue(g + 3, r0b, r1b, s0b, s1b)

        return carry

    lax.fori_loop(0, (_NCHUNK - 1) // 2, pair_body, 0)
    g_last = _NCHUNK - 1
    wait(g_last, r0a, r1a, s0a, s1a)
    compute(g_last, r0a, r1a)

    pltpu.sync_copy(ov, out.at[wid])


@jax.jit
def _run(x_disease, x_snorna, idx0, idx1):
    mesh = plsc.VectorSubcoreMesh(core_axis_name="c", subcore_axis_name="s")
    f = functools.partial(
        pl.kernel,
        mesh=mesh,
        out_type=jax.ShapeDtypeStruct((_NW, _PER_W), jnp.float32),
        scratch_types=[
            pltpu.VMEM((_NCHUNK, _C), jnp.int32),
            pltpu.VMEM((_NCHUNK, _C), jnp.int32),
            pltpu.VMEM((_C, D_FEAT), jnp.float32),
            pltpu.VMEM((_C, D_FEAT), jnp.float32),
            pltpu.VMEM((_C, D_FEAT), jnp.float32),
            pltpu.VMEM((_C, D_FEAT), jnp.float32),
            pltpu.VMEM((_PER_W,), jnp.float32),
            pltpu.SemaphoreType.DMA,
            pltpu.SemaphoreType.DMA,
            pltpu.SemaphoreType.DMA,
            pltpu.SemaphoreType.DMA,
        ],
        compiler_params=pltpu.CompilerParams(needs_layout_passes=False),
    )(_sc_kernel)
    return f(x_disease, x_snorna, idx0, idx1)


def kernel(x_disease, x_snorna, edge_label_index):
    idx0 = edge_label_index[0].reshape(_NW, _NCHUNK, _C)
    idx1 = edge_label_index[1].reshape(_NW, _NCHUNK, _C)
    return _run(x_disease, x_snorna, idx0, idx1).reshape(N_EDGES)


# speedup vs baseline: 2.3596x; 2.3596x over previous
import functools

import jax
import jax.numpy as jnp
from jax import lax
from jax.experimental import pallas as pl
from jax.experimental.pallas import tpu as pltpu
from jax.experimental.pallas import tpu_sc as plsc

N_NODES = 10000
D_FEAT = 128
N_EDGES = 320000

_NC = 2
_NS = 16
_NW = _NC * _NS
_PER_W = N_EDGES // _NW
_C = 80
_NCHUNK = _PER_W // _C

_LANES = 16
_KVEC = D_FEAT // _LANES


def _sc_kernel(xd, xs, idx0, idx1, out,
               i0all, i1all, r0a, r1a, r0b, r1b, ov, pv,
               s0a, s1a, s0b, s1b):
    wid = lax.axis_index("s") * _NC + lax.axis_index("c")
    pltpu.sync_copy(idx0.at[wid], i0all)
    pltpu.sync_copy(idx1.at[wid], i1all)

    lane = lax.iota(jnp.int32, _LANES)

    def issue(g, r0, r1, s0, s1):
        pltpu.async_copy(xd.at[i0all.at[g]], r0, s0)
        pltpu.async_copy(xs.at[i1all.at[g]], r1, s1)

    def wait(g, r0, r1, s0, s1):
        pltpu.make_async_copy(xd.at[i0all.at[g]], r0, s0).wait()
        pltpu.make_async_copy(xs.at[i1all.at[g]], r1, s1).wait()

    def compute(g, r0, r1):
        def group_body(gr, gcarry):
            for j in range(_LANES):
                e = gr * _LANES + j
                acc = r0[e, pl.ds(0, _LANES)] * r1[e, pl.ds(0, _LANES)]
                for k in range(1, _KVEC):
                    acc = acc + (r0[e, pl.ds(k * _LANES, _LANES)]
                                 * r1[e, pl.ds(k * _LANES, _LANES)])
                pv[j, pl.ds(0, _LANES)] = acc
            cols = [plsc.load_gather(pv, [lane, jnp.full((_LANES,), c, jnp.int32)])
                    for c in range(_LANES)]
            for step in (8, 4, 2, 1):
                cols = [cols[2 * t] + cols[2 * t + 1] for t in range(step)]
            ov[pl.ds(g * _C + gr * _LANES, _LANES)] = cols[0]
            return gcarry

        lax.fori_loop(0, _C // _LANES, group_body, 0)

    issue(0, r0a, r1a, s0a, s1a)
    issue(1, r0b, r1b, s0b, s1b)

    def pair_body(i, carry):
        g = 2 * i
        wait(g, r0a, r1a, s0a, s1a)
        compute(g, r0a, r1a)
        issue(g + 2, r0a, r1a, s0a, s1a)
        wait(g + 1, r0b, r1b, s0b, s1b)
        compute(g + 1, r0b, r1b)

        @pl.when(g + 3 < _NCHUNK)
        def _():
            issue(g + 3, r0b, r1b, s0b, s1b)

        return carry

    lax.fori_loop(0, (_NCHUNK - 1) // 2, pair_body, 0)
    g_last = _NCHUNK - 1
    wait(g_last, r0a, r1a, s0a, s1a)
    compute(g_last, r0a, r1a)

    pltpu.sync_copy(ov, out.at[wid])


@jax.jit
def _run(x_disease, x_snorna, idx0, idx1):
    mesh = plsc.VectorSubcoreMesh(core_axis_name="c", subcore_axis_name="s")
    f = functools.partial(
        pl.kernel,
        mesh=mesh,
        out_type=jax.ShapeDtypeStruct((_NW, _PER_W), jnp.float32),
        scratch_types=[
            pltpu.VMEM((_NCHUNK, _C), jnp.int32),
            pltpu.VMEM((_NCHUNK, _C), jnp.int32),
            pltpu.VMEM((_C, D_FEAT), jnp.float32),
            pltpu.VMEM((_C, D_FEAT), jnp.float32),
            pltpu.VMEM((_C, D_FEAT), jnp.float32),
            pltpu.VMEM((_C, D_FEAT), jnp.float32),
            pltpu.VMEM((_PER_W,), jnp.float32),
            pltpu.VMEM((_LANES, 40), jnp.float32),
            pltpu.SemaphoreType.DMA,
            pltpu.SemaphoreType.DMA,
            pltpu.SemaphoreType.DMA,
            pltpu.SemaphoreType.DMA,
        ],
        compiler_params=pltpu.CompilerParams(needs_layout_passes=False),
    )(_sc_kernel)
    return f(x_disease, x_snorna, idx0, idx1)


def kernel(x_disease, x_snorna, edge_label_index):
    idx0 = edge_label_index[0].reshape(_NW, _NCHUNK, _C)
    idx1 = edge_label_index[1].reshape(_NW, _NCHUNK, _C)
    return _run(x_disease, x_snorna, idx0, idx1).reshape(N_EDGES)


# speedup vs baseline: 2.4122x; 1.0223x over previous
import functools

import jax
import jax.numpy as jnp
from jax import lax
from jax.experimental import pallas as pl
from jax.experimental.pallas import tpu as pltpu
from jax.experimental.pallas import tpu_sc as plsc

N_NODES = 10000
D_FEAT = 128
N_EDGES = 320000

_NC = 2
_NS = 16
_NW = _NC * _NS
_PER_W = N_EDGES // _NW
_C = 80
_NCHUNK = _PER_W // _C

_LANES = 16
_KVEC = D_FEAT // _LANES


def _sc_kernel(xd, xs, idx0, idx1, out,
               i0all, i1all, r0a, r1a, r0b, r1b, ov, pv,
               s0a, s1a, s0b, s1b):
    wid = lax.axis_index("s") * _NC + lax.axis_index("c")
    pltpu.sync_copy(idx0.at[wid], i0all)
    pltpu.sync_copy(idx1.at[wid], i1all)

    lane = lax.iota(jnp.int32, _LANES)

    def issue(g, r0, r1, s0, s1):
        pltpu.async_copy(xd.at[i0all.at[g]], r0, s0)
        pltpu.async_copy(xs.at[i1all.at[g]], r1, s1)

    def wait(g, r0, r1, s0, s1):
        pltpu.make_async_copy(xd.at[i0all.at[g]], r0, s0).wait()
        pltpu.make_async_copy(xs.at[i1all.at[g]], r1, s1).wait()

    def compute(g, r0, r1):
        def group_body(gr, gcarry):
            for j in range(_LANES):
                e = gr * _LANES + j
                accs = []
                for k in range(_KVEC // 2):
                    a = plsc.bitcast(r0[e, pl.ds(k * _LANES, _LANES)],
                                     jnp.bfloat16)
                    b = plsc.bitcast(r1[e, pl.ds(k * _LANES, _LANES)],
                                     jnp.bfloat16)
                    a0, a1 = plsc.unpack(a, format=plsc.PackFormat.INTERLEAVED,
                                         preferred_element_type=jnp.float32)
                    b0, b1 = plsc.unpack(b, format=plsc.PackFormat.INTERLEAVED,
                                         preferred_element_type=jnp.float32)
                    accs.append(a0 * b0)
                    accs.append(a1 * b1)
                acc = ((accs[0] + accs[1]) + (accs[2] + accs[3])) + \
                      ((accs[4] + accs[5]) + (accs[6] + accs[7]))
                pv[j, pl.ds(0, _LANES)] = acc
            cols = [plsc.load_gather(pv, [lane, jnp.full((_LANES,), c, jnp.int32)])
                    for c in range(_LANES)]
            for step in (8, 4, 2, 1):
                cols = [cols[2 * t] + cols[2 * t + 1] for t in range(step)]
            ov[pl.ds(g * _C + gr * _LANES, _LANES)] = cols[0]
            return gcarry

        lax.fori_loop(0, _C // _LANES, group_body, 0)

    issue(0, r0a, r1a, s0a, s1a)
    issue(1, r0b, r1b, s0b, s1b)

    def pair_body(i, carry):
        g = 2 * i
        wait(g, r0a, r1a, s0a, s1a)
        compute(g, r0a, r1a)
        issue(g + 2, r0a, r1a, s0a, s1a)
        wait(g + 1, r0b, r1b, s0b, s1b)
        compute(g + 1, r0b, r1b)

        @pl.when(g + 3 < _NCHUNK)
        def _():
            issue(g + 3, r0b, r1b, s0b, s1b)

        return carry

    lax.fori_loop(0, (_NCHUNK - 1) // 2, pair_body, 0)
    g_last = _NCHUNK - 1
    wait(g_last, r0a, r1a, s0a, s1a)
    compute(g_last, r0a, r1a)

    pltpu.sync_copy(ov, out.at[wid])


@jax.jit
def _run(x_disease, x_snorna, idx0, idx1):
    mesh = plsc.VectorSubcoreMesh(core_axis_name="c", subcore_axis_name="s")
    f = functools.partial(
        pl.kernel,
        mesh=mesh,
        out_type=jax.ShapeDtypeStruct((_NW, _PER_W), jnp.float32),
        scratch_types=[
            pltpu.VMEM((_NCHUNK, _C), jnp.int32),
            pltpu.VMEM((_NCHUNK, _C), jnp.int32),
            pltpu.VMEM((_C, D_FEAT // 2), jnp.int32),
            pltpu.VMEM((_C, D_FEAT // 2), jnp.int32),
            pltpu.VMEM((_C, D_FEAT // 2), jnp.int32),
            pltpu.VMEM((_C, D_FEAT // 2), jnp.int32),
            pltpu.VMEM((_PER_W,), jnp.float32),
            pltpu.VMEM((_LANES, 40), jnp.float32),
            pltpu.SemaphoreType.DMA,
            pltpu.SemaphoreType.DMA,
            pltpu.SemaphoreType.DMA,
            pltpu.SemaphoreType.DMA,
        ],
        compiler_params=pltpu.CompilerParams(needs_layout_passes=False,
                                             use_tc_tiling_on_sc=False),
    )(_sc_kernel)
    return f(x_disease, x_snorna, idx0, idx1)


def kernel(x_disease, x_snorna, edge_label_index):
    idx0 = edge_label_index[0].reshape(_NW, _NCHUNK, _C)
    idx1 = edge_label_index[1].reshape(_NW, _NCHUNK, _C)
    xd = lax.bitcast_convert_type(
        x_disease.astype(jnp.bfloat16).reshape(N_NODES, D_FEAT // 2, 2),
        jnp.int32)
    xs = lax.bitcast_convert_type(
        x_snorna.astype(jnp.bfloat16).reshape(N_NODES, D_FEAT // 2, 2),
        jnp.int32)
    return _run(xd, xs, idx0, idx1).reshape(N_EDGES)
